# baseline (speedup 1.0000x reference)
import functools

import jax
import jax.numpy as jnp
from jax import lax
from jax.experimental import pallas as pl
from jax.experimental.pallas import tpu as pltpu
from jax.experimental.pallas import tpu_sc as plsc

N_NODES = 10000
N_EDGES = 320000
D_FEAT = 128
O_OUT = 64

NC = 2
NS = 16
NW = NC * NS
GRP = 16
E_PAD = 327680
N_ACC = 10112
ROWS_PER_TILE = N_ACC // NS
DEG_W = 16
AUG_W = D_FEAT + DEG_W


def _agg_body(width, wlen, *refs):
    (src_hbm, row_hbm, col_hbm, p_hbm,
     row_v, col_v, rows_a, rows_b, acc, sem_a, sem_b) = refs

    bufs = (rows_a, rows_b)
    sems = (sem_a, sem_b)
    core = lax.axis_index("core")
    sub = lax.axis_index("subcore")
    wpt = E_PAD // wlen // NW
    grps = wpt // GRP

    @pl.loop(0, wlen)
    def _(r):
        @pl.loop(0, width // 16)
        def _(j):
            rows_a[r, pl.ds(pl.multiple_of(j * 16, 16), 16)] = jnp.zeros(
                (16,), jnp.float32)

    r0 = sub * ROWS_PER_TILE
    chunks = ROWS_PER_TILE // wlen
    tail = ROWS_PER_TILE - chunks * wlen
    for k in range(chunks):
        pltpu.sync_copy(rows_a, acc.at[pl.ds(r0 + k * wlen, wlen)])
    pltpu.sync_copy(rows_a.at[pl.ds(0, tail)],
                    acc.at[pl.ds(r0 + chunks * wlen, tail)])

    plsc.subcore_barrier()

    base = (core * NS + sub) * wpt

    @pl.loop(0, grps)
    def _(g):
        blk = base + g * GRP
        pltpu.sync_copy(row_hbm.at[pl.ds(blk, GRP)], row_v)
        pltpu.sync_copy(col_hbm.at[pl.ds(blk, GRP)], col_v)
        copies = [pltpu.async_copy(src_hbm.at[row_v.at[0]], bufs[0], sems[0]),
                  None]
        for j in range(1, GRP):
            b = j % 2
            copies[b] = pltpu.async_copy(src_hbm.at[row_v.at[j]],
                                         bufs[b], sems[b])
            copies[1 - b].wait()
            pltpu.sync_copy(bufs[1 - b], acc.at[col_v.at[j - 1]], add=True)
        last = (GRP - 1) % 2
        copies[last].wait()
        pltpu.sync_copy(bufs[last], acc.at[col_v.at[GRP - 1]], add=True)

    plsc.subcore_barrier()

    for k in range(chunks):
        pltpu.sync_copy(acc.at[pl.ds(r0 + k * wlen, wlen)], rows_a)
        pltpu.sync_copy(rows_a, p_hbm.at[core, pl.ds(r0 + k * wlen, wlen)])
    pltpu.sync_copy(acc.at[pl.ds(r0 + chunks * wlen, tail)],
                    rows_a.at[pl.ds(0, tail)])
    pltpu.sync_copy(rows_a.at[pl.ds(0, tail)],
                    p_hbm.at[core, pl.ds(r0 + chunks * wlen, tail)])


def _sc_aggregate(src, row2, col2, width, wlen):
    mesh = plsc.VectorSubcoreMesh(core_axis_name="core",
                                  subcore_axis_name="subcore")
    body = functools.partial(_agg_body, width, wlen)
    cp = pltpu.CompilerParams(use_tc_tiling_on_sc=False)
    return pl.kernel(
        body, mesh=mesh, compiler_params=cp,
        out_type=jax.ShapeDtypeStruct((NC, N_ACC, width), jnp.float32),
        scratch_types=[pltpu.VMEM((GRP, wlen), jnp.int32),
                       pltpu.VMEM((GRP, wlen), jnp.int32),
                       pltpu.VMEM((wlen, width), jnp.float32),
                       pltpu.VMEM((wlen, width), jnp.float32),
                       pltpu.VMEM_SHARED((N_ACC, width), jnp.float32),
                       pltpu.SemaphoreType.DMA,
                       pltpu.SemaphoreType.DMA])(src, row2, col2)


ROW_BLK = 2000


def _combine_body(p_ref, x_ref, m_ref, bc_ref, n1_ref, y_ref):
    deg = p_ref[0, :, D_FEAT:D_FEAT + 1] + p_ref[1, :, D_FEAT:D_FEAT + 1]
    dinv = 1.0 / jnp.maximum(deg, 1.0)
    n1 = (p_ref[0, :, :D_FEAT] + p_ref[1, :, :D_FEAT]) * dinv
    n1_ref[...] = n1
    h = jnp.dot(x_ref[...], m_ref[0], preferred_element_type=jnp.float32)
    h = h + jnp.dot(n1, m_ref[1], preferred_element_type=jnp.float32)
    y_ref[...] = h + bc_ref[...]


def _combine(p, x, m, bc):
    grid = (N_NODES // ROW_BLK,)
    return pl.pallas_call(
        _combine_body,
        grid=grid,
        in_specs=[
            pl.BlockSpec((NC, ROW_BLK, AUG_W), lambda i: (0, i, 0)),
            pl.BlockSpec((ROW_BLK, D_FEAT), lambda i: (i, 0)),
            pl.BlockSpec((3, D_FEAT, D_FEAT), lambda i: (0, 0, 0)),
            pl.BlockSpec((1, D_FEAT), lambda i: (0, 0)),
        ],
        out_specs=[pl.BlockSpec((ROW_BLK, D_FEAT), lambda i: (i, 0)),
                   pl.BlockSpec((ROW_BLK, D_FEAT), lambda i: (i, 0))],
        out_shape=[jax.ShapeDtypeStruct((N_NODES, D_FEAT), jnp.float32),
                   jax.ShapeDtypeStruct((N_NODES, D_FEAT), jnp.float32)],
    )(p, x, m, bc)


def _final_body(y_ref, q_ref, degp_ref, m_ref, wo_ref, bo_ref, out_ref):
    deg = (degp_ref[0, :, D_FEAT:D_FEAT + 1]
           + degp_ref[1, :, D_FEAT:D_FEAT + 1])
    dinv = 1.0 / jnp.maximum(deg, 1.0)
    n2 = (q_ref[0] + q_ref[1]) * dinv
    h = y_ref[...] + jnp.dot(n2, m_ref[2], preferred_element_type=jnp.float32)
    h = jnp.maximum(h, 0.0)
    out_ref[...] = (jnp.dot(h, wo_ref[...], preferred_element_type=jnp.float32)
                    + bo_ref[...])


def _final(y, q, p, m, wo, bo):
    grid = (N_NODES // ROW_BLK,)
    return pl.pallas_call(
        _final_body,
        grid=grid,
        in_specs=[
            pl.BlockSpec((ROW_BLK, D_FEAT), lambda i: (i, 0)),
            pl.BlockSpec((NC, ROW_BLK, D_FEAT), lambda i: (0, i, 0)),
            pl.BlockSpec((NC, ROW_BLK, AUG_W), lambda i: (0, i, 0)),
            pl.BlockSpec((3, D_FEAT, D_FEAT), lambda i: (0, 0, 0)),
            pl.BlockSpec((D_FEAT, O_OUT), lambda i: (0, 0)),
            pl.BlockSpec((1, O_OUT), lambda i: (0, 0)),
        ],
        out_specs=pl.BlockSpec((ROW_BLK, O_OUT), lambda i: (i, 0)),
        out_shape=jax.ShapeDtypeStruct((N_NODES, O_OUT), jnp.float32),
    )(y, q, p, m, wo, bo)


def kernel(x, edge_index, W_ego, b_ego, W_n1, b_n1, W_n2, b_n2,
           W_comb, b_comb, W_out, b_out):
    row = edge_index[0]
    col = edge_index[1]

    pad = E_PAD - N_EDGES
    ar = jnp.arange(pad, dtype=jnp.int32)
    rowp = jnp.concatenate([row, ar % N_NODES])
    colp = jnp.concatenate([col, N_NODES + (ar % (N_ACC - N_NODES))])

    xa = jnp.concatenate(
        [x, jnp.ones((N_NODES, DEG_W), dtype=jnp.float32)], axis=1)

    m = jnp.stack([W_ego @ W_comb[:D_FEAT],
                   W_n1 @ W_comb[D_FEAT:2 * D_FEAT],
                   W_n2 @ W_comb[2 * D_FEAT:]], axis=0)
    bc = (b_ego @ W_comb[:D_FEAT] + b_n1 @ W_comb[D_FEAT:2 * D_FEAT]
          + b_n2 @ W_comb[2 * D_FEAT:] + b_comb)[None, :]

    row2 = rowp.reshape(-1, 80)
    col2 = colp.reshape(-1, 80)
    p = _sc_aggregate(xa, row2, col2, width=AUG_W, wlen=80)
    n1, y = _combine(p, x, m, bc)
    q = _sc_aggregate(n1, row2, col2, width=D_FEAT, wlen=80)
    return _final(y, q, p, m, W_out, b_out[None, :])


# speedup vs baseline: 1.0442x; 1.0442x over previous
import functools

import jax
import jax.numpy as jnp
from jax import lax
from jax.experimental import pallas as pl
from jax.experimental.pallas import tpu as pltpu
from jax.experimental.pallas import tpu_sc as plsc

N_NODES = 10000
N_EDGES = 320000
D_FEAT = 128
O_OUT = 64

NC = 2
NS = 16
NW = NC * NS
GRP = 16
E_PAD = 327680
N_ACC = 10112
ROWS_PER_TILE = N_ACC // NS
DEG_W = 16
AUG_W = D_FEAT + DEG_W


def _agg_body(width, wlen, *refs):
    (src_hbm, row_hbm, col_hbm, p_hbm,
     row_v, col_v, rows_a, rows_b, acc, sem_a, sem_b) = refs

    bufs = (rows_a, rows_b)
    sems = (sem_a, sem_b)
    core = lax.axis_index("core")
    sub = lax.axis_index("subcore")
    wpt = E_PAD // wlen // NW
    grps = wpt // GRP

    @pl.loop(0, wlen)
    def _(r):
        @pl.loop(0, width // 16)
        def _(j):
            rows_a[r, pl.ds(pl.multiple_of(j * 16, 16), 16)] = jnp.zeros(
                (16,), jnp.float32)

    r0 = sub * ROWS_PER_TILE
    chunks = ROWS_PER_TILE // wlen
    tail = ROWS_PER_TILE - chunks * wlen
    for k in range(chunks):
        pltpu.sync_copy(rows_a, acc.at[pl.ds(r0 + k * wlen, wlen)])
    pltpu.sync_copy(rows_a.at[pl.ds(0, tail)],
                    acc.at[pl.ds(r0 + chunks * wlen, tail)])

    plsc.subcore_barrier()

    base = (core * NS + sub) * wpt

    @pl.loop(0, grps)
    def _(g):
        blk = base + g * GRP
        pltpu.sync_copy(row_hbm.at[pl.ds(blk, GRP)], row_v)
        pltpu.sync_copy(col_hbm.at[pl.ds(blk, GRP)], col_v)
        copies = [pltpu.async_copy(src_hbm.at[row_v.at[0]], bufs[0], sems[0]),
                  None]
        for j in range(1, GRP):
            b = j % 2
            copies[b] = pltpu.async_copy(src_hbm.at[row_v.at[j]],
                                         bufs[b], sems[b])
            copies[1 - b].wait()
            pltpu.sync_copy(bufs[1 - b], acc.at[col_v.at[j - 1]], add=True)
        last = (GRP - 1) % 2
        copies[last].wait()
        pltpu.sync_copy(bufs[last], acc.at[col_v.at[GRP - 1]], add=True)

    plsc.subcore_barrier()

    for k in range(chunks):
        pltpu.sync_copy(acc.at[pl.ds(r0 + k * wlen, wlen)], rows_a)
        pltpu.sync_copy(rows_a, p_hbm.at[core, pl.ds(r0 + k * wlen, wlen)])
    pltpu.sync_copy(acc.at[pl.ds(r0 + chunks * wlen, tail)],
                    rows_a.at[pl.ds(0, tail)])
    pltpu.sync_copy(rows_a.at[pl.ds(0, tail)],
                    p_hbm.at[core, pl.ds(r0 + chunks * wlen, tail)])


def _sc_aggregate(src, row2, col2, width, wlen):
    mesh = plsc.VectorSubcoreMesh(core_axis_name="core",
                                  subcore_axis_name="subcore")
    body = functools.partial(_agg_body, width, wlen)
    cp = None
    if width % 128 != 0:
        cp = pltpu.CompilerParams(use_tc_tiling_on_sc=False)
    return pl.kernel(
        body, mesh=mesh, compiler_params=cp,
        out_type=jax.ShapeDtypeStruct((NC, N_ACC, width), jnp.float32),
        scratch_types=[pltpu.VMEM((GRP, wlen), jnp.int32),
                       pltpu.VMEM((GRP, wlen), jnp.int32),
                       pltpu.VMEM((wlen, width), jnp.float32),
                       pltpu.VMEM((wlen, width), jnp.float32),
                       pltpu.VMEM_SHARED((N_ACC, width), jnp.float32),
                       pltpu.SemaphoreType.DMA,
                       pltpu.SemaphoreType.DMA])(src, row2, col2)


ROW_BLK = 2000


def _combine_body(p_ref, x_ref, m_ref, bc_ref, n1_ref, y_ref):
    deg = p_ref[0, :, D_FEAT:D_FEAT + 1] + p_ref[1, :, D_FEAT:D_FEAT + 1]
    dinv = 1.0 / jnp.maximum(deg, 1.0)
    n1 = (p_ref[0, :, :D_FEAT] + p_ref[1, :, :D_FEAT]) * dinv
    n1_ref[...] = n1
    h = jnp.dot(x_ref[...], m_ref[0], preferred_element_type=jnp.float32)
    h = h + jnp.dot(n1, m_ref[1], preferred_element_type=jnp.float32)
    y_ref[...] = h + bc_ref[...]


def _combine(p, x, m, bc):
    grid = (N_NODES // ROW_BLK,)
    return pl.pallas_call(
        _combine_body,
        grid=grid,
        in_specs=[
            pl.BlockSpec((NC, ROW_BLK, AUG_W), lambda i: (0, i, 0)),
            pl.BlockSpec((ROW_BLK, D_FEAT), lambda i: (i, 0)),
            pl.BlockSpec((3, D_FEAT, D_FEAT), lambda i: (0, 0, 0)),
            pl.BlockSpec((1, D_FEAT), lambda i: (0, 0)),
        ],
        out_specs=[pl.BlockSpec((ROW_BLK, D_FEAT), lambda i: (i, 0)),
                   pl.BlockSpec((ROW_BLK, D_FEAT), lambda i: (i, 0))],
        out_shape=[jax.ShapeDtypeStruct((N_NODES, D_FEAT), jnp.float32),
                   jax.ShapeDtypeStruct((N_NODES, D_FEAT), jnp.float32)],
    )(p, x, m, bc)


def _final_body(y_ref, q_ref, degp_ref, m_ref, wo_ref, bo_ref, out_ref):
    deg = (degp_ref[0, :, D_FEAT:D_FEAT + 1]
           + degp_ref[1, :, D_FEAT:D_FEAT + 1])
    dinv = 1.0 / jnp.maximum(deg, 1.0)
    n2 = (q_ref[0] + q_ref[1]) * dinv
    h = y_ref[...] + jnp.dot(n2, m_ref[2], preferred_element_type=jnp.float32)
    h = jnp.maximum(h, 0.0)
    out_ref[...] = (jnp.dot(h, wo_ref[...], preferred_element_type=jnp.float32)
                    + bo_ref[...])


def _final(y, q, p, m, wo, bo):
    grid = (N_NODES // ROW_BLK,)
    return pl.pallas_call(
        _final_body,
        grid=grid,
        in_specs=[
            pl.BlockSpec((ROW_BLK, D_FEAT), lambda i: (i, 0)),
            pl.BlockSpec((NC, ROW_BLK, D_FEAT), lambda i: (0, i, 0)),
            pl.BlockSpec((NC, ROW_BLK, AUG_W), lambda i: (0, i, 0)),
            pl.BlockSpec((3, D_FEAT, D_FEAT), lambda i: (0, 0, 0)),
            pl.BlockSpec((D_FEAT, O_OUT), lambda i: (0, 0)),
            pl.BlockSpec((1, O_OUT), lambda i: (0, 0)),
        ],
        out_specs=pl.BlockSpec((ROW_BLK, O_OUT), lambda i: (i, 0)),
        out_shape=jax.ShapeDtypeStruct((N_NODES, O_OUT), jnp.float32),
    )(y, q, p, m, wo, bo)


def kernel(x, edge_index, W_ego, b_ego, W_n1, b_n1, W_n2, b_n2,
           W_comb, b_comb, W_out, b_out):
    row = edge_index[0]
    col = edge_index[1]

    pad = E_PAD - N_EDGES
    ar = jnp.arange(pad, dtype=jnp.int32)
    rowp = jnp.concatenate([row, ar % N_NODES])
    colp = jnp.concatenate([col, N_NODES + (ar % (N_ACC - N_NODES))])

    xa = jnp.concatenate(
        [x, jnp.ones((N_NODES, DEG_W), dtype=jnp.float32)], axis=1)

    m = jnp.stack([W_ego @ W_comb[:D_FEAT],
                   W_n1 @ W_comb[D_FEAT:2 * D_FEAT],
                   W_n2 @ W_comb[2 * D_FEAT:]], axis=0)
    bc = (b_ego @ W_comb[:D_FEAT] + b_n1 @ W_comb[D_FEAT:2 * D_FEAT]
          + b_n2 @ W_comb[2 * D_FEAT:] + b_comb)[None, :]

    p = _sc_aggregate(xa, rowp.reshape(-1, 80), colp.reshape(-1, 80),
                      width=AUG_W, wlen=80)
    n1, y = _combine(p, x, m, bc)
    q = _sc_aggregate(n1, rowp.reshape(-1, 128), colp.reshape(-1, 128),
                      width=D_FEAT, wlen=128)
    return _final(y, q, p, m, W_out, b_out[None, :])


# speedup vs baseline: 1.1029x; 1.0563x over previous
import functools

import jax
import jax.numpy as jnp
from jax import lax
from jax.experimental import pallas as pl
from jax.experimental.pallas import tpu as pltpu
from jax.experimental.pallas import tpu_sc as plsc

N_NODES = 10000
N_EDGES = 320000
D_FEAT = 128
O_OUT = 64

NC = 2
NS = 16
NW = NC * NS
GRP = 32
NBUF = 3
E_PAD = 327680
N_ACC = 10112
ROWS_PER_TILE = N_ACC // NS
DEG_W = 16
AUG_W = D_FEAT + DEG_W


def _agg_body(width, wlen, *refs):
    (src_hbm, row_hbm, col_hbm, p_hbm, row_v, col_v) = refs[:6]
    bufs = refs[6:6 + NBUF]
    acc = refs[6 + NBUF]
    gsems = refs[7 + NBUF:7 + 2 * NBUF]
    ssems = refs[7 + 2 * NBUF:7 + 3 * NBUF]
    rows_a = bufs[0]

    core = lax.axis_index("core")
    sub = lax.axis_index("subcore")
    wpt = E_PAD // wlen // NW
    grps = wpt // GRP

    @pl.loop(0, wlen)
    def _(r):
        @pl.loop(0, width // 16)
        def _(j):
            rows_a[r, pl.ds(pl.multiple_of(j * 16, 16), 16)] = jnp.zeros(
                (16,), jnp.float32)

    r0 = sub * ROWS_PER_TILE
    chunks = ROWS_PER_TILE // wlen
    tail = ROWS_PER_TILE - chunks * wlen
    for k in range(chunks):
        pltpu.sync_copy(rows_a, acc.at[pl.ds(r0 + k * wlen, wlen)])
    pltpu.sync_copy(rows_a.at[pl.ds(0, tail)],
                    acc.at[pl.ds(r0 + chunks * wlen, tail)])

    plsc.subcore_barrier()

    base = (core * NS + sub) * wpt

    @pl.loop(0, grps)
    def _(g):
        blk = base + g * GRP
        pltpu.sync_copy(row_hbm.at[pl.ds(blk, GRP)], row_v)
        pltpu.sync_copy(col_hbm.at[pl.ds(blk, GRP)], col_v)
        cg = [None] * NBUF
        cs = [None] * NBUF
        for j in range(GRP):
            b = j % NBUF
            if j >= NBUF:
                cs[b].wait()
            cg[b] = pltpu.async_copy(src_hbm.at[row_v.at[j]],
                                     bufs[b], gsems[b])
            if j >= 1:
                pb = (j - 1) % NBUF
                cg[pb].wait()
                cs[pb] = pltpu.async_copy(bufs[pb],
                                          acc.at[col_v.at[j - 1]],
                                          ssems[pb], add=True)
        lb = (GRP - 1) % NBUF
        cg[lb].wait()
        cs[lb] = pltpu.async_copy(bufs[lb], acc.at[col_v.at[GRP - 1]],
                                  ssems[lb], add=True)
        for b in range(NBUF):
            cs[b].wait()

    plsc.subcore_barrier()

    for k in range(chunks):
        pltpu.sync_copy(acc.at[pl.ds(r0 + k * wlen, wlen)], rows_a)
        pltpu.sync_copy(rows_a, p_hbm.at[core, pl.ds(r0 + k * wlen, wlen)])
    pltpu.sync_copy(acc.at[pl.ds(r0 + chunks * wlen, tail)],
                    rows_a.at[pl.ds(0, tail)])
    pltpu.sync_copy(rows_a.at[pl.ds(0, tail)],
                    p_hbm.at[core, pl.ds(r0 + chunks * wlen, tail)])


def _sc_aggregate(src, row2, col2, width, wlen):
    mesh = plsc.VectorSubcoreMesh(core_axis_name="core",
                                  subcore_axis_name="subcore")
    body = functools.partial(_agg_body, width, wlen)
    cp = None
    if width % 128 != 0:
        cp = pltpu.CompilerParams(use_tc_tiling_on_sc=False)
    scratch = ([pltpu.VMEM((GRP, wlen), jnp.int32),
                pltpu.VMEM((GRP, wlen), jnp.int32)]
               + [pltpu.VMEM((wlen, width), jnp.float32)] * NBUF
               + [pltpu.VMEM_SHARED((N_ACC, width), jnp.float32)]
               + [pltpu.SemaphoreType.DMA] * (2 * NBUF))
    return pl.kernel(
        body, mesh=mesh, compiler_params=cp,
        out_type=jax.ShapeDtypeStruct((NC, N_ACC, width), jnp.float32),
        scratch_types=scratch)(src, row2, col2)


ROW_BLK = 2000


def _combine_body(p_ref, x_ref, m_ref, bc_ref, n1_ref, y_ref):
    deg = p_ref[0, :, D_FEAT:D_FEAT + 1] + p_ref[1, :, D_FEAT:D_FEAT + 1]
    dinv = 1.0 / jnp.maximum(deg, 1.0)
    n1 = (p_ref[0, :, :D_FEAT] + p_ref[1, :, :D_FEAT]) * dinv
    n1_ref[...] = n1
    h = jnp.dot(x_ref[...], m_ref[0], preferred_element_type=jnp.float32)
    h = h + jnp.dot(n1, m_ref[1], preferred_element_type=jnp.float32)
    y_ref[...] = h + bc_ref[...]


def _combine(p, x, m, bc):
    grid = (N_NODES // ROW_BLK,)
    return pl.pallas_call(
        _combine_body,
        grid=grid,
        in_specs=[
            pl.BlockSpec((NC, ROW_BLK, AUG_W), lambda i: (0, i, 0)),
            pl.BlockSpec((ROW_BLK, D_FEAT), lambda i: (i, 0)),
            pl.BlockSpec((3, D_FEAT, D_FEAT), lambda i: (0, 0, 0)),
            pl.BlockSpec((1, D_FEAT), lambda i: (0, 0)),
        ],
        out_specs=[pl.BlockSpec((ROW_BLK, D_FEAT), lambda i: (i, 0)),
                   pl.BlockSpec((ROW_BLK, D_FEAT), lambda i: (i, 0))],
        out_shape=[jax.ShapeDtypeStruct((N_NODES, D_FEAT), jnp.float32),
                   jax.ShapeDtypeStruct((N_NODES, D_FEAT), jnp.float32)],
    )(p, x, m, bc)


def _final_body(y_ref, q_ref, degp_ref, m_ref, wo_ref, bo_ref, out_ref):
    deg = (degp_ref[0, :, D_FEAT:D_FEAT + 1]
           + degp_ref[1, :, D_FEAT:D_FEAT + 1])
    dinv = 1.0 / jnp.maximum(deg, 1.0)
    n2 = (q_ref[0] + q_ref[1]) * dinv
    h = y_ref[...] + jnp.dot(n2, m_ref[2], preferred_element_type=jnp.float32)
    h = jnp.maximum(h, 0.0)
    out_ref[...] = (jnp.dot(h, wo_ref[...], preferred_element_type=jnp.float32)
                    + bo_ref[...])


def _final(y, q, p, m, wo, bo):
    grid = (N_NODES // ROW_BLK,)
    return pl.pallas_call(
        _final_body,
        grid=grid,
        in_specs=[
            pl.BlockSpec((ROW_BLK, D_FEAT), lambda i: (i, 0)),
            pl.BlockSpec((NC, ROW_BLK, D_FEAT), lambda i: (0, i, 0)),
            pl.BlockSpec((NC, ROW_BLK, AUG_W), lambda i: (0, i, 0)),
            pl.BlockSpec((3, D_FEAT, D_FEAT), lambda i: (0, 0, 0)),
            pl.BlockSpec((D_FEAT, O_OUT), lambda i: (0, 0)),
            pl.BlockSpec((1, O_OUT), lambda i: (0, 0)),
        ],
        out_specs=pl.BlockSpec((ROW_BLK, O_OUT), lambda i: (i, 0)),
        out_shape=jax.ShapeDtypeStruct((N_NODES, O_OUT), jnp.float32),
    )(y, q, p, m, wo, bo)


def kernel(x, edge_index, W_ego, b_ego, W_n1, b_n1, W_n2, b_n2,
           W_comb, b_comb, W_out, b_out):
    row = edge_index[0]
    col = edge_index[1]

    pad = E_PAD - N_EDGES
    ar = jnp.arange(pad, dtype=jnp.int32)
    rowp = jnp.concatenate([row, ar % N_NODES])
    colp = jnp.concatenate([col, N_NODES + (ar % (N_ACC - N_NODES))])

    xa = jnp.concatenate(
        [x, jnp.ones((N_NODES, DEG_W), dtype=jnp.float32)], axis=1)

    m = jnp.stack([W_ego @ W_comb[:D_FEAT],
                   W_n1 @ W_comb[D_FEAT:2 * D_FEAT],
                   W_n2 @ W_comb[2 * D_FEAT:]], axis=0)
    bc = (b_ego @ W_comb[:D_FEAT] + b_n1 @ W_comb[D_FEAT:2 * D_FEAT]
          + b_n2 @ W_comb[2 * D_FEAT:] + b_comb)[None, :]

    row2 = rowp.reshape(-1, 64)
    col2 = colp.reshape(-1, 64)
    p = _sc_aggregate(xa, row2, col2, width=AUG_W, wlen=64)
    n1, y = _combine(p, x, m, bc)
    q = _sc_aggregate(n1, row2, col2, width=D_FEAT, wlen=64)
    return _final(y, q, p, m, W_out, b_out[None, :])


# speedup vs baseline: 1.1106x; 1.0069x over previous
import functools

import jax
import jax.numpy as jnp
from jax import lax
from jax.experimental import pallas as pl
from jax.experimental.pallas import tpu as pltpu
from jax.experimental.pallas import tpu_sc as plsc

N_NODES = 10000
N_EDGES = 320000
D_FEAT = 128
O_OUT = 64

NC = 2
NS = 16
NW = NC * NS
GRP = 32
NBUF = 3
E_PAD = 327680
N_ACC = 10112
ROWS_PER_TILE = N_ACC // NS
DEG_W = 16
AUG_W = D_FEAT + DEG_W


def _agg_body(width, wlen, nbuf, lag, *refs):
    (src_hbm, row_hbm, col_hbm, p_hbm, row_v, col_v) = refs[:6]
    bufs = refs[6:6 + nbuf]
    acc = refs[6 + nbuf]
    gsems = refs[7 + nbuf:7 + 2 * nbuf]
    ssems = refs[7 + 2 * nbuf:7 + 3 * nbuf]
    rows_a = bufs[0]

    core = lax.axis_index("core")
    sub = lax.axis_index("subcore")
    wpt = E_PAD // wlen // NW
    grps = wpt // GRP

    @pl.loop(0, wlen)
    def _(r):
        @pl.loop(0, width // 16)
        def _(j):
            rows_a[r, pl.ds(pl.multiple_of(j * 16, 16), 16)] = jnp.zeros(
                (16,), jnp.float32)

    r0 = sub * ROWS_PER_TILE
    chunks = ROWS_PER_TILE // wlen
    tail = ROWS_PER_TILE - chunks * wlen
    for k in range(chunks):
        pltpu.sync_copy(rows_a, acc.at[pl.ds(r0 + k * wlen, wlen)])
    pltpu.sync_copy(rows_a.at[pl.ds(0, tail)],
                    acc.at[pl.ds(r0 + chunks * wlen, tail)])

    plsc.subcore_barrier()

    base = (core * NS + sub) * wpt

    @pl.loop(0, grps)
    def _(g):
        blk = base + g * GRP
        pltpu.sync_copy(row_hbm.at[pl.ds(blk, GRP)], row_v)
        pltpu.sync_copy(col_hbm.at[pl.ds(blk, GRP)], col_v)
        cg = [None] * nbuf
        cs = [None] * nbuf
        for j in range(GRP):
            b = j % nbuf
            if j >= nbuf:
                cs[b].wait()
            cg[b] = pltpu.async_copy(src_hbm.at[row_v.at[j]],
                                     bufs[b], gsems[b])
            if j >= lag:
                pb = (j - lag) % nbuf
                cg[pb].wait()
                cs[pb] = pltpu.async_copy(bufs[pb],
                                          acc.at[col_v.at[j - lag]],
                                          ssems[pb], add=True)
        for t in range(GRP - lag, GRP):
            pb = t % nbuf
            cg[pb].wait()
            cs[pb] = pltpu.async_copy(bufs[pb], acc.at[col_v.at[t]],
                                      ssems[pb], add=True)
        for b in range(nbuf):
            cs[b].wait()

    plsc.subcore_barrier()

    for k in range(chunks):
        pltpu.sync_copy(acc.at[pl.ds(r0 + k * wlen, wlen)], rows_a)
        pltpu.sync_copy(rows_a, p_hbm.at[core, pl.ds(r0 + k * wlen, wlen)])
    pltpu.sync_copy(acc.at[pl.ds(r0 + chunks * wlen, tail)],
                    rows_a.at[pl.ds(0, tail)])
    pltpu.sync_copy(rows_a.at[pl.ds(0, tail)],
                    p_hbm.at[core, pl.ds(r0 + chunks * wlen, tail)])


def _sc_aggregate(src, row2, col2, width, wlen, nbuf, lag):
    mesh = plsc.VectorSubcoreMesh(core_axis_name="core",
                                  subcore_axis_name="subcore")
    body = functools.partial(_agg_body, width, wlen, nbuf, lag)
    cp = None
    if width % 128 != 0:
        cp = pltpu.CompilerParams(use_tc_tiling_on_sc=False)
    scratch = ([pltpu.VMEM((GRP, wlen), jnp.int32),
                pltpu.VMEM((GRP, wlen), jnp.int32)]
               + [pltpu.VMEM((wlen, width), jnp.float32)] * nbuf
               + [pltpu.VMEM_SHARED((N_ACC, width), jnp.float32)]
               + [pltpu.SemaphoreType.DMA] * (2 * nbuf))
    return pl.kernel(
        body, mesh=mesh, compiler_params=cp,
        out_type=jax.ShapeDtypeStruct((NC, N_ACC, width), jnp.float32),
        scratch_types=scratch)(src, row2, col2)


ROW_BLK = 2000


def _combine_body(p_ref, x_ref, m_ref, bc_ref, n1_ref, y_ref):
    deg = p_ref[0, :, D_FEAT:D_FEAT + 1] + p_ref[1, :, D_FEAT:D_FEAT + 1]
    dinv = 1.0 / jnp.maximum(deg, 1.0)
    n1 = (p_ref[0, :, :D_FEAT] + p_ref[1, :, :D_FEAT]) * dinv
    n1_ref[...] = n1
    h = jnp.dot(x_ref[...], m_ref[0], preferred_element_type=jnp.float32)
    h = h + jnp.dot(n1, m_ref[1], preferred_element_type=jnp.float32)
    y_ref[...] = h + bc_ref[...]


def _combine(p, x, m, bc):
    grid = (N_NODES // ROW_BLK,)
    return pl.pallas_call(
        _combine_body,
        grid=grid,
        in_specs=[
            pl.BlockSpec((NC, ROW_BLK, AUG_W), lambda i: (0, i, 0)),
            pl.BlockSpec((ROW_BLK, D_FEAT), lambda i: (i, 0)),
            pl.BlockSpec((3, D_FEAT, D_FEAT), lambda i: (0, 0, 0)),
            pl.BlockSpec((1, D_FEAT), lambda i: (0, 0)),
        ],
        out_specs=[pl.BlockSpec((ROW_BLK, D_FEAT), lambda i: (i, 0)),
                   pl.BlockSpec((ROW_BLK, D_FEAT), lambda i: (i, 0))],
        out_shape=[jax.ShapeDtypeStruct((N_NODES, D_FEAT), jnp.float32),
                   jax.ShapeDtypeStruct((N_NODES, D_FEAT), jnp.float32)],
    )(p, x, m, bc)


def _final_body(y_ref, q_ref, degp_ref, m_ref, wo_ref, bo_ref, out_ref):
    deg = (degp_ref[0, :, D_FEAT:D_FEAT + 1]
           + degp_ref[1, :, D_FEAT:D_FEAT + 1])
    dinv = 1.0 / jnp.maximum(deg, 1.0)
    n2 = (q_ref[0] + q_ref[1]) * dinv
    h = y_ref[...] + jnp.dot(n2, m_ref[2], preferred_element_type=jnp.float32)
    h = jnp.maximum(h, 0.0)
    out_ref[...] = (jnp.dot(h, wo_ref[...], preferred_element_type=jnp.float32)
                    + bo_ref[...])


def _final(y, q, p, m, wo, bo):
    grid = (N_NODES // ROW_BLK,)
    return pl.pallas_call(
        _final_body,
        grid=grid,
        in_specs=[
            pl.BlockSpec((ROW_BLK, D_FEAT), lambda i: (i, 0)),
            pl.BlockSpec((NC, ROW_BLK, D_FEAT), lambda i: (0, i, 0)),
            pl.BlockSpec((NC, ROW_BLK, AUG_W), lambda i: (0, i, 0)),
            pl.BlockSpec((3, D_FEAT, D_FEAT), lambda i: (0, 0, 0)),
            pl.BlockSpec((D_FEAT, O_OUT), lambda i: (0, 0)),
            pl.BlockSpec((1, O_OUT), lambda i: (0, 0)),
        ],
        out_specs=pl.BlockSpec((ROW_BLK, O_OUT), lambda i: (i, 0)),
        out_shape=jax.ShapeDtypeStruct((N_NODES, O_OUT), jnp.float32),
    )(y, q, p, m, wo, bo)


def kernel(x, edge_index, W_ego, b_ego, W_n1, b_n1, W_n2, b_n2,
           W_comb, b_comb, W_out, b_out):
    row = edge_index[0]
    col = edge_index[1]

    pad = E_PAD - N_EDGES
    ar = jnp.arange(pad, dtype=jnp.int32)
    rowp = jnp.concatenate([row, ar % N_NODES])
    colp = jnp.concatenate([col, N_NODES + (ar % (N_ACC - N_NODES))])

    xa = jnp.concatenate(
        [x, jnp.ones((N_NODES, DEG_W), dtype=jnp.float32)], axis=1)

    m = jnp.stack([W_ego @ W_comb[:D_FEAT],
                   W_n1 @ W_comb[D_FEAT:2 * D_FEAT],
                   W_n2 @ W_comb[2 * D_FEAT:]], axis=0)
    bc = (b_ego @ W_comb[:D_FEAT] + b_n1 @ W_comb[D_FEAT:2 * D_FEAT]
          + b_n2 @ W_comb[2 * D_FEAT:] + b_comb)[None, :]

    row2 = rowp.reshape(-1, 64)
    col2 = colp.reshape(-1, 64)
    p = _sc_aggregate(xa, row2, col2, width=AUG_W, wlen=64, nbuf=3, lag=1)
    n1, y = _combine(p, x, m, bc)
    q = _sc_aggregate(n1, row2, col2, width=D_FEAT, wlen=64, nbuf=4, lag=2)
    return _final(y, q, p, m, W_out, b_out[None, :])
